# baseline (speedup 1.0000x reference)
import functools

import jax
import jax.numpy as jnp
from jax import lax
from jax.experimental import pallas as pl
from jax.experimental.pallas import tpu as pltpu
from jax.experimental.pallas import tpu_sc as plsc

DIM = 64
RPB = 128
K = 8
NC = 2
NS = 16
NW = NC * NS


@functools.partial(jax.jit, static_argnums=(2,))
def _emb_gather(idx2d, table, outer):
    nrows = idx2d.shape[0]
    rows_per_w = nrows // NW

    mesh = plsc.VectorSubcoreMesh(core_axis_name="c", subcore_axis_name="s")

    @functools.partial(
        pl.kernel,
        mesh=mesh,
        out_type=jax.ShapeDtypeStruct((nrows, RPB, DIM), jnp.float32),
        scratch_types=[
            pltpu.VMEM((K, RPB), jnp.int32),
            pltpu.VMEM((K, RPB, DIM), jnp.float32),
            pltpu.SemaphoreType.DMA,
        ],
        compiler_params=pltpu.CompilerParams(use_tc_tiling_on_sc=False),
    )
    def gather_kernel(idx_hbm, table_hbm, out_hbm, idx_v, rows_v, sem):
        wid = lax.axis_index("s") * NC + lax.axis_index("c")
        base = wid * rows_per_w

        def body(g, carry):
            rb = base + g * K
            pltpu.sync_copy(idx_hbm.at[pl.ds(rb, K)], idx_v)
            copies = [
                pltpu.async_copy(table_hbm.at[idx_v.at[j]], rows_v.at[j], sem)
                for j in range(K)
            ]
            for c in copies:
                c.wait()
            pltpu.sync_copy(rows_v, out_hbm.at[pl.ds(rb, K)])
            return carry

        lax.fori_loop(0, outer, body, 0)

    return gather_kernel(idx2d, table)


def kernel(tokens, emb_weight):
    b, l = tokens.shape
    n = b * l
    nrows = n // RPB
    outer = nrows // NW // K
    idx2d = tokens.reshape(nrows, RPB)
    out = _emb_gather(idx2d, emb_weight, outer)
    return out.reshape(b, l, DIM)


# speedup vs baseline: 1.0148x; 1.0148x over previous
import functools

import jax
import jax.numpy as jnp
from jax import lax
from jax.experimental import pallas as pl
from jax.experimental.pallas import tpu as pltpu
from jax.experimental.pallas import tpu_sc as plsc

DIM = 64
RPB = 128
K = 5
NB = 2
NC = 2
NS = 16
NW = NC * NS


@functools.partial(jax.jit, static_argnums=(2,))
def _emb_gather(idx2d, table, rows_per_w):
    nrows = idx2d.shape[0]
    steps = rows_per_w // K

    mesh = plsc.VectorSubcoreMesh(core_axis_name="c", subcore_axis_name="s")

    @functools.partial(
        pl.kernel,
        mesh=mesh,
        out_type=jax.ShapeDtypeStruct((nrows, RPB, DIM), jnp.float32),
        scratch_types=[
            pltpu.VMEM((rows_per_w, RPB), jnp.int32),
            pltpu.VMEM((NB, K, RPB, DIM), jnp.float32),
            pltpu.SemaphoreType.DMA,
            pltpu.SemaphoreType.DMA,
            pltpu.SemaphoreType.DMA,
        ],
        compiler_params=pltpu.CompilerParams(use_tc_tiling_on_sc=False),
    )
    def gather_kernel(idx_hbm, table_hbm, out_hbm, idx_v, rows_v, gsem,
                      osem0, osem1):
        wid = lax.axis_index("s") * NC + lax.axis_index("c")
        base = wid * rows_per_w
        osems = (osem0, osem1)

        pltpu.sync_copy(idx_hbm.at[pl.ds(base, rows_per_w)], idx_v)

        def fire(s, b):
            return [
                pltpu.async_copy(
                    table_hbm.at[idx_v.at[s * K + j]],
                    rows_v.at[b, j], gsem)
                for j in range(K)
            ]

        def store(s, b):
            pltpu.async_copy(rows_v.at[b], out_hbm.at[pl.ds(base + s * K, K)],
                             osems[b])

        def drain_store(b):
            pltpu.make_async_copy(
                rows_v.at[b], out_hbm.at[pl.ds(base, K)], osems[b]).wait()

        g0 = fire(0, 0)
        g1 = fire(1, 1)
        for c in g0:
            c.wait()
        store(0, 0)
        for c in g1:
            c.wait()
        store(1, 1)

        def body(i, carry):
            s0 = 2 * i
            drain_store(0)
            c0 = fire(s0, 0)
            drain_store(1)
            c1 = fire(s0 + 1, 1)
            for c in c0:
                c.wait()
            store(s0, 0)
            for c in c1:
                c.wait()
            store(s0 + 1, 1)
            return carry

        lax.fori_loop(1, steps // 2, body, 0)

        drain_store(0)
        drain_store(1)

    return gather_kernel(idx2d, table)


def kernel(tokens, emb_weight):
    b, l = tokens.shape
    n = b * l
    nrows = n // RPB
    rows_per_w = nrows // NW
    idx2d = tokens.reshape(nrows, RPB)
    out = _emb_gather(idx2d, emb_weight, rows_per_w)
    return out.reshape(b, l, DIM)


# speedup vs baseline: 1.1464x; 1.1297x over previous
import functools

import jax
import jax.numpy as jnp
from jax import lax
from jax.experimental import pallas as pl
from jax.experimental.pallas import tpu as pltpu
from jax.experimental.pallas import tpu_sc as plsc

DIM = 64
RPB = 128
K = 5
NB = 2
NC = 2
NS = 16
NW = NC * NS
PACK_C = 2048


def _pack_body(t2_ref, z_ref, scr):
    t = t2_ref[...].T
    scr[:, 0:64] = t
    ev = scr[0::2, :]
    od = scr[1::2, :]
    odr = pltpu.roll(od, 64, 1)
    lane = lax.broadcasted_iota(jnp.int32, (PACK_C // 2, 128), 1)
    z_ref[...] = jnp.where(lane < 64, ev, odr)


def _pack_table(t2):
    vocab = t2.shape[1]
    return pl.pallas_call(
        _pack_body,
        grid=(pl.cdiv(vocab, PACK_C),),
        in_specs=[pl.BlockSpec((DIM, PACK_C), lambda j: (0, j))],
        out_specs=pl.BlockSpec((PACK_C // 2, 128), lambda j: (j, 0)),
        out_shape=jax.ShapeDtypeStruct((vocab // 2, 128), jnp.float32),
        scratch_shapes=[pltpu.VMEM((PACK_C, 128), jnp.float32)],
    )(t2)


@functools.partial(jax.jit, static_argnums=(2,))
def _emb_gather(idx2d, table, rows_per_w):
    nrows = idx2d.shape[0]
    steps = rows_per_w // K

    mesh = plsc.VectorSubcoreMesh(core_axis_name="c", subcore_axis_name="s")

    @functools.partial(
        pl.kernel,
        mesh=mesh,
        out_type=jax.ShapeDtypeStruct((nrows, RPB, DIM), jnp.float32),
        scratch_types=[
            pltpu.VMEM((rows_per_w, RPB), jnp.int32),
            pltpu.VMEM((NB, K, RPB, DIM), jnp.float32),
            pltpu.SemaphoreType.DMA,
            pltpu.SemaphoreType.DMA,
            pltpu.SemaphoreType.DMA,
        ],
        compiler_params=pltpu.CompilerParams(use_tc_tiling_on_sc=False),
    )
    def gather_kernel(idx_hbm, table_hbm, out_hbm, idx_v, rows_v, gsem,
                      osem0, osem1):
        wid = lax.axis_index("s") * NC + lax.axis_index("c")
        base = wid * rows_per_w
        osems = (osem0, osem1)

        pltpu.sync_copy(idx_hbm.at[pl.ds(base, rows_per_w)], idx_v)

        def fire(s, b):
            return [
                pltpu.async_copy(
                    table_hbm.at[idx_v.at[s * K + j]],
                    rows_v.at[b, j], gsem)
                for j in range(K)
            ]

        def store(s, b):
            pltpu.async_copy(rows_v.at[b], out_hbm.at[pl.ds(base + s * K, K)],
                             osems[b])

        def drain_store(b):
            pltpu.make_async_copy(
                rows_v.at[b], out_hbm.at[pl.ds(base, K)], osems[b]).wait()

        g0 = fire(0, 0)
        g1 = fire(1, 1)
        for c in g0:
            c.wait()
        store(0, 0)
        for c in g1:
            c.wait()
        store(1, 1)

        def body(i, carry):
            s0 = 2 * i
            drain_store(0)
            c0 = fire(s0, 0)
            drain_store(1)
            c1 = fire(s0 + 1, 1)
            for c in c0:
                c.wait()
            store(s0, 0)
            for c in c1:
                c.wait()
            store(s0 + 1, 1)
            return carry

        lax.fori_loop(1, steps // 2, body, 0)

        drain_store(0)
        drain_store(1)

    return gather_kernel(idx2d, table)


def kernel(tokens, emb_weight):
    b, l = tokens.shape
    n = b * l
    vocab, dim = emb_weight.shape
    nrows = n // RPB
    rows_per_w = nrows // NW
    z = _pack_table(emb_weight.T)
    table_lin = z.reshape(vocab, dim)
    idx2d = tokens.reshape(nrows, RPB)
    out = _emb_gather(idx2d, table_lin, rows_per_w)
    return out.reshape(b, l, DIM)
